# Initial kernel scaffold; baseline (speedup 1.0000x reference)
import jax
import jax.numpy as jnp
from jax.experimental import pallas as pl


def kernel(ans_emb, prev_inds, ocr_embedding, pos_table, type_table, ans_ln_w, ans_ln_b, ocr_ln_w, ocr_ln_b, emb_ln_w, emb_ln_b):
    raise NotImplementedError("write your pallas kernel here")


# speedup vs baseline: 1.0230x; 1.0230x over previous
import functools

import jax
import jax.numpy as jnp
from jax import lax
from jax.experimental import pallas as pl
from jax.experimental.pallas import tpu as pltpu
from jax.experimental.pallas import tpu_sc as plsc

H = 768
V = 5000
B = 128
S = 32
OCR = 50
L = 16
NCH = H // L
NC = 2
NS = 16
NW = NC * NS
B_PER_W = B // NW
EPS = 1e-5
_MAGIC = 0x5F3759DF


def _prep_body(pos_ref, type_ref, embw_ref, embb_ref, ansb_ref, ocrb_ref, out_ref):
    x = pos_ref[...] + type_ref[0, :][None, :]
    mu = jnp.mean(x, axis=-1, keepdims=True)
    var = jnp.mean((x - mu) ** 2, axis=-1, keepdims=True)
    pn = (x - mu) * lax.rsqrt(var + EPS) * embw_ref[...][None, :] + embb_ref[...][None, :]
    out_ref[0, :, :] = pn + ansb_ref[...][None, :]
    out_ref[1, :, :] = pn + ocrb_ref[...][None, :]


def _prep_posb(pos32, type0, emb_w, emb_b, ans_b, ocr_b):
    return pl.pallas_call(
        _prep_body,
        out_shape=jax.ShapeDtypeStruct((2, S, H), jnp.float32),
    )(pos32, type0, emb_w, emb_b, ans_b, ocr_b)


def _rsqrt_nr(t):
    seed = jnp.full((L,), _MAGIC, jnp.int32) - lax.shift_right_logical(
        plsc.bitcast(t, jnp.int32), 1)
    y = plsc.bitcast(seed, jnp.float32)
    half = t * 0.5
    for _ in range(3):
        y = y * (1.5 - half * y * y)
    return y


def _sc_body(ans_hbm, prev_hbm, ocr_hbm, posb_hbm, wab_hbm, out_hbm,
             idx_v, aidx_v, oidx_v, rows2, outv, posb_v, wab_v, sem_a, sem_o):
    cid = lax.axis_index("c")
    sid = lax.axis_index("s")
    wid = sid * NC + cid
    pltpu.sync_copy(posb_hbm, posb_v)
    pltpu.sync_copy(wab_hbm, wab_v)
    lane_iota = lax.iota(jnp.int32, 16)

    def batch_body(i, carry):
        b = wid * B_PER_W + i
        pltpu.sync_copy(prev_hbm.at[b], idx_v)
        for c in range(S // L):
            ii = idx_v[pl.ds(c * L, L)]
            aidx_v[pl.ds(c * L, L)] = jnp.minimum(ii, V - 1)
            oo = jnp.clip(ii - V, 0, OCR - 1) + b * OCR
            oidx_v[pl.ds(c * L, L)] = oo
        cp_a = pltpu.async_copy(ans_hbm.at[aidx_v], rows2.at[0], sem_a)
        cp_o = pltpu.async_copy(ocr_hbm.at[oidx_v], rows2.at[1], sem_o)
        cp_a.wait()
        cp_o.wait()

        def row_body(r, carry2):
            chunk = r // L
            lane = r - chunk * L
            iv = idx_v[pl.ds(chunk * L, L)].astype(jnp.float32)
            idx_rf = jnp.sum(jnp.where(lane_iota == lane, iv, 0.0))
            is_ocr = jnp.where(idx_rf >= V, 1, 0)
            def stat_body(c, acc):
                s1, s2 = acc
                x = rows2[is_ocr, r, pl.ds(c * L, L)]
                return (s1 + x, s2 + x * x)
            zero = jnp.zeros((L,), jnp.float32)
            s1, s2 = lax.fori_loop(0, NCH, stat_body, (zero, zero))
            tot = jnp.sum(s1)
            tot2 = jnp.sum(s2)
            mean = tot * (1.0 / H)
            var = tot2 * (1.0 / H) - mean * mean
            meanv = jnp.full((L,), mean, jnp.float32)
            invv = _rsqrt_nr(jnp.full((L,), var + EPS, jnp.float32))
            w_off = is_ocr * H

            def out_body(c, _):
                x = rows2[is_ocr, r, pl.ds(c * L, L)]
                wv = wab_v[pl.ds(w_off + c * L, L)]
                pb = posb_v[is_ocr, r, pl.ds(c * L, L)]
                outv[r, pl.ds(c * L, L)] = (x - meanv) * invv * wv + pb
                return 0
            lax.fori_loop(0, NCH, out_body, 0)
            return carry2

        lax.fori_loop(0, S, row_body, 0)
        pltpu.sync_copy(outv, out_hbm.at[b])
        return carry

    lax.fori_loop(0, B_PER_W, batch_body, 0)


_sc_kernel = functools.partial(
    pl.kernel,
    out_type=jax.ShapeDtypeStruct((B, S, H), jnp.float32),
    mesh=plsc.VectorSubcoreMesh(core_axis_name="c", subcore_axis_name="s"),
    compiler_params=pltpu.CompilerParams(needs_layout_passes=False),
    scratch_types=[
        pltpu.VMEM((S,), jnp.int32),
        pltpu.VMEM((S,), jnp.int32),
        pltpu.VMEM((S,), jnp.int32),
        pltpu.VMEM((2, S, H), jnp.float32),
        pltpu.VMEM((S, H), jnp.float32),
        pltpu.VMEM((2, S, H), jnp.float32),
        pltpu.VMEM((2 * H,), jnp.float32),
        pltpu.SemaphoreType.DMA,
        pltpu.SemaphoreType.DMA,
    ],
)(_sc_body)


def kernel(ans_emb, prev_inds, ocr_embedding, pos_table, type_table,
           ans_ln_w, ans_ln_b, ocr_ln_w, ocr_ln_b, emb_ln_w, emb_ln_b):
    posb = _prep_posb(pos_table[:S], type_table[0:1], emb_ln_w, emb_ln_b,
                      ans_ln_b, ocr_ln_b)
    wab = jnp.concatenate([ans_ln_w, ocr_ln_w])
    ocr_flat = ocr_embedding.reshape(B * OCR, H)
    return _sc_kernel(ans_emb, prev_inds, ocr_flat, posb, wab)


# speedup vs baseline: 1.6497x; 1.6126x over previous
import functools

import jax
import jax.numpy as jnp
from jax import lax
from jax.experimental import pallas as pl
from jax.experimental.pallas import tpu as pltpu
from jax.experimental.pallas import tpu_sc as plsc

H = 768
V = 5000
B = 128
S = 32
OCR = 50
L = 16
NCH = H // L
NC = 2
NS = 16
NW = NC * NS
B_PER_W = B // NW
EPS = 1e-5
_MAGIC = 0x5F3759DF


def _prep_body(pos_ref, type_ref, embw_ref, embb_ref, ansb_ref, ocrb_ref, out_ref):
    x = pos_ref[...] + type_ref[0, :][None, :]
    mu = jnp.mean(x, axis=-1, keepdims=True)
    var = jnp.mean((x - mu) ** 2, axis=-1, keepdims=True)
    pn = (x - mu) * lax.rsqrt(var + EPS) * embw_ref[...][None, :] + embb_ref[...][None, :]
    out_ref[0, :, :] = pn + ansb_ref[...][None, :]
    out_ref[1, :, :] = pn + ocrb_ref[...][None, :]


def _prep_posb(pos32, type0, emb_w, emb_b, ans_b, ocr_b):
    return pl.pallas_call(
        _prep_body,
        out_shape=jax.ShapeDtypeStruct((2, S, H), jnp.float32),
    )(pos32, type0, emb_w, emb_b, ans_b, ocr_b)


def _rsqrt_nr(t):
    seed = jnp.full((L,), _MAGIC, jnp.int32) - lax.shift_right_logical(
        plsc.bitcast(t, jnp.int32), 1)
    y = plsc.bitcast(seed, jnp.float32)
    half = t * 0.5
    for _ in range(3):
        y = y * (1.5 - half * y * y)
    return y


def _sc_body(ans_hbm, prev_hbm, ocr_hbm, posb_hbm, wab_hbm, out_hbm,
             idx_v, aidx_v, oidx_v, rows2, outv, posb_v, wab_v, sem_a, sem_o):
    cid = lax.axis_index("c")
    sid = lax.axis_index("s")
    wid = sid * NC + cid
    pltpu.sync_copy(posb_hbm, posb_v)
    pltpu.sync_copy(wab_hbm, wab_v)
    lane_iota = lax.iota(jnp.int32, 16)

    def batch_body(i, carry):
        b = wid * B_PER_W + i
        pltpu.sync_copy(prev_hbm.at[b], idx_v)
        for c in range(S // L):
            ii = idx_v[pl.ds(c * L, L)]
            aidx_v[pl.ds(c * L, L)] = jnp.minimum(ii, V - 1)
            oidx_v[pl.ds(c * L, L)] = jnp.clip(ii - V, 0, OCR - 1)
        cp_a = pltpu.async_copy(ans_hbm.at[aidx_v], rows2.at[0], sem_a)
        cp_o = pltpu.async_copy(ocr_hbm.at[b].at[oidx_v], rows2.at[1], sem_o)
        cp_a.wait()
        cp_o.wait()

        @plsc.parallel_loop(0, S, 1)
        def row_body(r):
            chunk = r // L
            lane = r - chunk * L
            iv = idx_v[pl.ds(chunk * L, L)].astype(jnp.float32)
            idx_rf = jnp.sum(jnp.where(lane_iota == lane, iv, 0.0))
            is_ocr = jnp.where(idx_rf >= V, 1, 0)
            zero = jnp.zeros((L,), jnp.float32)

            @plsc.parallel_loop(0, NCH, 4, carry=(zero, zero, zero, zero,
                                                  zero, zero, zero, zero))
            def stat_body(c, acc):
                a0, a1, a2, a3, q0, q1, q2, q3 = acc
                x0 = rows2[is_ocr, r, pl.ds(c * L, L)]
                x1 = rows2[is_ocr, r, pl.ds((c + 1) * L, L)]
                x2 = rows2[is_ocr, r, pl.ds((c + 2) * L, L)]
                x3 = rows2[is_ocr, r, pl.ds((c + 3) * L, L)]
                return (a0 + x0, a1 + x1, a2 + x2, a3 + x3,
                        q0 + x0 * x0, q1 + x1 * x1, q2 + x2 * x2, q3 + x3 * x3)
            a0, a1, a2, a3, q0, q1, q2, q3 = stat_body
            tot = jnp.sum((a0 + a1) + (a2 + a3))
            tot2 = jnp.sum((q0 + q1) + (q2 + q3))
            mean = tot * (1.0 / H)
            var = tot2 * (1.0 / H) - mean * mean
            meanv = jnp.full((L,), mean, jnp.float32)
            invv = _rsqrt_nr(jnp.full((L,), var + EPS, jnp.float32))
            w_off = is_ocr * H

            @plsc.parallel_loop(0, NCH, 1, unroll=4)
            def out_body(c):
                x = rows2[is_ocr, r, pl.ds(c * L, L)]
                wv = wab_v[pl.ds(w_off + c * L, L)]
                pb = posb_v[is_ocr, r, pl.ds(c * L, L)]
                outv[r, pl.ds(c * L, L)] = (x - meanv) * invv * wv + pb

        pltpu.sync_copy(outv, out_hbm.at[b])
        return carry

    lax.fori_loop(0, B_PER_W, batch_body, 0)


_sc_kernel = functools.partial(
    pl.kernel,
    out_type=jax.ShapeDtypeStruct((B, S, H), jnp.float32),
    mesh=plsc.VectorSubcoreMesh(core_axis_name="c", subcore_axis_name="s"),
    compiler_params=pltpu.CompilerParams(needs_layout_passes=False),
    scratch_types=[
        pltpu.VMEM((S,), jnp.int32),
        pltpu.VMEM((S,), jnp.int32),
        pltpu.VMEM((S,), jnp.int32),
        pltpu.VMEM((2, S, H), jnp.float32),
        pltpu.VMEM((S, H), jnp.float32),
        pltpu.VMEM((2, S, H), jnp.float32),
        pltpu.VMEM((2 * H,), jnp.float32),
        pltpu.SemaphoreType.DMA,
        pltpu.SemaphoreType.DMA,
    ],
)(_sc_body)


def kernel(ans_emb, prev_inds, ocr_embedding, pos_table, type_table,
           ans_ln_w, ans_ln_b, ocr_ln_w, ocr_ln_b, emb_ln_w, emb_ln_b):
    posb = _prep_posb(pos_table[:S], type_table[0:1], emb_ln_w, emb_ln_b,
                      ans_ln_b, ocr_ln_b)
    wab = jnp.concatenate([ans_ln_w, ocr_ln_w])
    return _sc_kernel(ans_emb, prev_inds, ocr_embedding, posb, wab)


# speedup vs baseline: 1.9798x; 1.2001x over previous
import functools

import jax
import jax.numpy as jnp
from jax import lax
from jax.experimental import pallas as pl
from jax.experimental.pallas import tpu as pltpu
from jax.experimental.pallas import tpu_sc as plsc

H = 768
V = 5000
B = 128
S = 32
OCR = 50
L = 16
NCH = H // L
NC = 2
NS = 16
NW = NC * NS
B_PER_W = B // NW
EPS = 1e-5
_MAGIC = 0x5F3759DF


def _prep_body(pos_ref, type_ref, embw_ref, embb_ref, ansb_ref, ocrb_ref, out_ref):
    x = pos_ref[...] + type_ref[0, :][None, :]
    mu = jnp.mean(x, axis=-1, keepdims=True)
    var = jnp.mean((x - mu) ** 2, axis=-1, keepdims=True)
    pn = (x - mu) * lax.rsqrt(var + EPS) * embw_ref[...][None, :] + embb_ref[...][None, :]
    out_ref[0, :, :] = pn + ansb_ref[...][None, :]
    out_ref[1, :, :] = pn + ocrb_ref[...][None, :]


def _prep_posb(pos32, type0, emb_w, emb_b, ans_b, ocr_b):
    return pl.pallas_call(
        _prep_body,
        out_shape=jax.ShapeDtypeStruct((2, S, H), jnp.float32),
    )(pos32, type0, emb_w, emb_b, ans_b, ocr_b)


def _rsqrt_nr(t):
    seed = jnp.full((L,), _MAGIC, jnp.int32) - lax.shift_right_logical(
        plsc.bitcast(t, jnp.int32), 1)
    y = plsc.bitcast(seed, jnp.float32)
    half = t * 0.5
    for _ in range(3):
        y = y * (1.5 - half * y * y)
    return y


NH = 2 * B_PER_W
HR = S // 2


def _sc_body(ans_hbm, prev_hbm, ocr_hbm, posb_hbm, wab_hbm, out_hbm,
             idx_all, aidx_v, oidx_v, rows_a, rows_b, out_a, out_b,
             posb_v, wab_v, sem_i, sem_g0, sem_g1, sem_w0, sem_w1):
    cid = lax.axis_index("c")
    sid = lax.axis_index("s")
    wid = sid * NC + cid
    pltpu.sync_copy(prev_hbm.at[pl.ds(wid * B_PER_W, B_PER_W)], idx_all)
    pltpu.sync_copy(posb_hbm, posb_v)
    pltpu.sync_copy(wab_hbm, wab_v)
    lane_iota = lax.iota(jnp.int32, 16)

    for g in range(NH):
        ii = idx_all[g // 2, pl.ds((g % 2) * HR, HR)]
        aidx_v[pl.ds(g * HR, HR)] = jnp.minimum(ii, V - 1)
        oidx_v[pl.ds(g * HR, HR)] = jnp.clip(ii - V, 0, OCR - 1)

    rows_bufs = (rows_a, rows_b)
    out_bufs = (out_a, out_b)
    gather_sems = (sem_g0, sem_g1)
    write_sems = (sem_w0, sem_w1)

    def issue_gather(g):
        buf = rows_bufs[g % 2]
        sem = gather_sems[g % 2]
        b = wid * B_PER_W + g // 2
        cpa = pltpu.async_copy(ans_hbm.at[aidx_v.at[pl.ds(g * HR, HR)]],
                               buf.at[0], sem)
        cpo = pltpu.async_copy(ocr_hbm.at[b].at[oidx_v.at[pl.ds(g * HR, HR)]],
                               buf.at[1], sem)
        return (cpa, cpo)

    gathers = {0: issue_gather(0), 1: issue_gather(1)}
    writes = {}
    for g in range(NH):
        buf = rows_bufs[g % 2]
        outv = out_bufs[g % 2]
        if g - 2 in writes:
            writes[g - 2].wait()
        cpa, cpo = gathers[g]
        cpa.wait()
        cpo.wait()
        iv_f = idx_all[g // 2, pl.ds((g % 2) * HR, HR)].astype(jnp.float32)
        s_base = (g % 2) * HR

        @plsc.parallel_loop(0, HR, 1)
        def row_body(r):
            idx_rf = jnp.sum(jnp.where(lane_iota == r, iv_f, 0.0))
            is_ocr = jnp.where(idx_rf >= V, 1, 0)
            zero = jnp.zeros((L,), jnp.float32)

            @plsc.parallel_loop(0, NCH, 4, carry=(zero, zero, zero, zero,
                                                  zero, zero, zero, zero))
            def stat_body(c, acc):
                a0, a1, a2, a3, q0, q1, q2, q3 = acc
                x0 = buf[is_ocr, r, pl.ds(c * L, L)]
                x1 = buf[is_ocr, r, pl.ds((c + 1) * L, L)]
                x2 = buf[is_ocr, r, pl.ds((c + 2) * L, L)]
                x3 = buf[is_ocr, r, pl.ds((c + 3) * L, L)]
                return (a0 + x0, a1 + x1, a2 + x2, a3 + x3,
                        q0 + x0 * x0, q1 + x1 * x1, q2 + x2 * x2, q3 + x3 * x3)
            a0, a1, a2, a3, q0, q1, q2, q3 = stat_body
            tot = jnp.sum((a0 + a1) + (a2 + a3))
            tot2 = jnp.sum((q0 + q1) + (q2 + q3))
            mean = tot * (1.0 / H)
            var = tot2 * (1.0 / H) - mean * mean
            meanv = jnp.full((L,), mean, jnp.float32)
            invv = _rsqrt_nr(jnp.full((L,), var + EPS, jnp.float32))
            w_off = is_ocr * H

            @plsc.parallel_loop(0, NCH, 1, unroll=4)
            def out_body(c):
                x = buf[is_ocr, r, pl.ds(c * L, L)]
                wv = wab_v[pl.ds(w_off + c * L, L)]
                pb = posb_v[is_ocr, s_base + r, pl.ds(c * L, L)]
                outv[r, pl.ds(c * L, L)] = (x - meanv) * invv * wv + pb

        b = wid * B_PER_W + g // 2
        writes[g] = pltpu.async_copy(
            outv, out_hbm.at[b, pl.ds(s_base, HR)], write_sems[g % 2])
        if g + 2 < NH:
            gathers[g + 2] = issue_gather(g + 2)
    writes[NH - 2].wait()
    writes[NH - 1].wait()


_sc_kernel = functools.partial(
    pl.kernel,
    out_type=jax.ShapeDtypeStruct((B, S, H), jnp.float32),
    mesh=plsc.VectorSubcoreMesh(core_axis_name="c", subcore_axis_name="s"),
    compiler_params=pltpu.CompilerParams(needs_layout_passes=False),
    scratch_types=[
        pltpu.VMEM((B_PER_W, S), jnp.int32),
        pltpu.VMEM((NH * HR,), jnp.int32),
        pltpu.VMEM((NH * HR,), jnp.int32),
        pltpu.VMEM((2, HR, H), jnp.float32),
        pltpu.VMEM((2, HR, H), jnp.float32),
        pltpu.VMEM((HR, H), jnp.float32),
        pltpu.VMEM((HR, H), jnp.float32),
        pltpu.VMEM((2, S, H), jnp.float32),
        pltpu.VMEM((2 * H,), jnp.float32),
        pltpu.SemaphoreType.DMA,
        pltpu.SemaphoreType.DMA,
        pltpu.SemaphoreType.DMA,
        pltpu.SemaphoreType.DMA,
        pltpu.SemaphoreType.DMA,
    ],
)(_sc_body)


def kernel(ans_emb, prev_inds, ocr_embedding, pos_table, type_table,
           ans_ln_w, ans_ln_b, ocr_ln_w, ocr_ln_b, emb_ln_w, emb_ln_b):
    posb = _prep_posb(pos_table[:S], type_table[0:1], emb_ln_w, emb_ln_b,
                      ans_ln_b, ocr_ln_b)
    wab = jnp.concatenate([ans_ln_w, ocr_ln_w])
    return _sc_kernel(ans_emb, prev_inds, ocr_embedding, posb, wab)


# speedup vs baseline: 2.8220x; 1.4254x over previous
import functools

import jax
import jax.numpy as jnp
from jax import lax
from jax.experimental import pallas as pl
from jax.experimental.pallas import tpu as pltpu
from jax.experimental.pallas import tpu_sc as plsc

H = 768
V = 5000
B = 128
S = 32
OCR = 50
L = 16
NCH = H // L
NC = 2
NS = 16
NW = NC * NS
B_PER_W = B // NW
EPS = 1e-5
_MAGIC = 0x5F3759DF


def _prep_body(pos_ref, type_ref, embw_ref, embb_ref, ansb_ref, ocrb_ref,
               answ_ref, ocrw_ref, posb_ref, wab_ref):
    x = pos_ref[0:S, :] + type_ref[0, :][None, :]
    mu = jnp.mean(x, axis=-1, keepdims=True)
    var = jnp.mean((x - mu) ** 2, axis=-1, keepdims=True)
    pn = (x - mu) * lax.rsqrt(var + EPS) * embw_ref[...][None, :] + embb_ref[...][None, :]
    posb_ref[0, :, :] = pn + ansb_ref[...][None, :]
    posb_ref[1, :, :] = pn + ocrb_ref[...][None, :]
    wab_ref[pl.ds(0, H)] = answ_ref[...]
    wab_ref[pl.ds(H, H)] = ocrw_ref[...]


def _prep_posb(pos_table, type_table, emb_w, emb_b, ans_b, ocr_b, ans_w, ocr_w):
    return pl.pallas_call(
        _prep_body,
        out_shape=(jax.ShapeDtypeStruct((2, S, H), jnp.float32),
                   jax.ShapeDtypeStruct((2 * H,), jnp.float32)),
    )(pos_table, type_table, emb_w, emb_b, ans_b, ocr_b, ans_w, ocr_w)


def _rsqrt_nr(t):
    seed = jnp.full((L,), _MAGIC, jnp.int32) - lax.shift_right_logical(
        plsc.bitcast(t, jnp.int32), 1)
    y = plsc.bitcast(seed, jnp.float32)
    half = t * 0.5
    for _ in range(3):
        y = y * (1.5 - half * y * y)
    return y


NH = 2 * B_PER_W
HR = S // 2


def _sc_body(ans_hbm, prev_hbm, ocr_hbm, posb_hbm, wab_hbm, out_hbm,
             idx_sb, idx_v, aidx_v, oidx_v, rows_a, rows_b, out_a, out_b,
             posb_v, wab_v, sem_i, sem_g0, sem_g1, sem_w0, sem_w1):
    cid = lax.axis_index("c")
    sid = lax.axis_index("s")
    wid = sid * NC + cid
    pltpu.sync_copy(prev_hbm, idx_sb)
    pltpu.sync_copy(posb_hbm, posb_v)
    pltpu.sync_copy(wab_hbm, wab_v)
    lane_iota = lax.iota(jnp.int32, 16)

    for g in range(NH):
        b = wid * B_PER_W + g // 2
        row_ids = lane_iota + (g % 2) * HR
        col_ids = jnp.full((L,), b, jnp.int32)
        ii = plsc.load_gather(idx_sb, [row_ids, col_ids])
        idx_v[pl.ds(g * HR, HR)] = ii
        aidx_v[pl.ds(g * HR, HR)] = jnp.minimum(ii, V - 1)
        oidx_v[pl.ds(g * HR, HR)] = jnp.clip(ii - V, 0, OCR - 1) * B + b

    rows_bufs = (rows_a, rows_b)
    out_bufs = (out_a, out_b)
    gather_sems = (sem_g0, sem_g1)
    write_sems = (sem_w0, sem_w1)

    def issue_gather(g):
        buf = rows_bufs[g % 2]
        sem = gather_sems[g % 2]
        cpa = pltpu.async_copy(ans_hbm.at[aidx_v.at[pl.ds(g * HR, HR)]],
                               buf.at[0], sem)
        cpo = pltpu.async_copy(ocr_hbm.at[oidx_v.at[pl.ds(g * HR, HR)]],
                               buf.at[1], sem)
        return (cpa, cpo)

    gathers = {0: issue_gather(0), 1: issue_gather(1)}
    writes = {}
    for g in range(NH):
        buf = rows_bufs[g % 2]
        outv = out_bufs[g % 2]
        if g - 2 in writes:
            writes[g - 2].wait()
        cpa, cpo = gathers[g]
        cpa.wait()
        cpo.wait()
        iv_f = idx_v[pl.ds(g * HR, HR)].astype(jnp.float32)
        s_base = (g % 2) * HR

        @plsc.parallel_loop(0, HR, 1)
        def row_body(r):
            idx_rf = jnp.sum(jnp.where(lane_iota == r, iv_f, 0.0))
            is_ocr = jnp.where(idx_rf >= V, 1, 0)
            zero = jnp.zeros((L,), jnp.float32)

            @plsc.parallel_loop(0, NCH, 4, carry=(zero, zero, zero, zero,
                                                  zero, zero, zero, zero))
            def stat_body(c, acc):
                a0, a1, a2, a3, q0, q1, q2, q3 = acc
                x0 = buf[is_ocr, r, pl.ds(c * L, L)]
                x1 = buf[is_ocr, r, pl.ds((c + 1) * L, L)]
                x2 = buf[is_ocr, r, pl.ds((c + 2) * L, L)]
                x3 = buf[is_ocr, r, pl.ds((c + 3) * L, L)]
                return (a0 + x0, a1 + x1, a2 + x2, a3 + x3,
                        q0 + x0 * x0, q1 + x1 * x1, q2 + x2 * x2, q3 + x3 * x3)
            a0, a1, a2, a3, q0, q1, q2, q3 = stat_body
            tot = jnp.sum((a0 + a1) + (a2 + a3))
            tot2 = jnp.sum((q0 + q1) + (q2 + q3))
            mean = tot * (1.0 / H)
            var = tot2 * (1.0 / H) - mean * mean
            meanv = jnp.full((L,), mean, jnp.float32)
            invv = _rsqrt_nr(jnp.full((L,), var + EPS, jnp.float32))
            w_off = is_ocr * H

            @plsc.parallel_loop(0, NCH, 1, unroll=4)
            def out_body(c):
                x = buf[is_ocr, r, pl.ds(c * L, L)]
                wv = wab_v[pl.ds(w_off + c * L, L)]
                pb = posb_v[is_ocr, s_base + r, pl.ds(c * L, L)]
                outv[r, pl.ds(c * L, L)] = (x - meanv) * invv * wv + pb

        b = wid * B_PER_W + g // 2
        writes[g] = pltpu.async_copy(
            outv, out_hbm.at[b, pl.ds(s_base, HR)], write_sems[g % 2])
        if g + 2 < NH:
            gathers[g + 2] = issue_gather(g + 2)
    writes[NH - 2].wait()
    writes[NH - 1].wait()


_sc_kernel = functools.partial(
    pl.kernel,
    out_type=jax.ShapeDtypeStruct((B, S, H), jnp.float32),
    mesh=plsc.VectorSubcoreMesh(core_axis_name="c", subcore_axis_name="s"),
    compiler_params=pltpu.CompilerParams(needs_layout_passes=False),
    scratch_types=[
        pltpu.VMEM((S, B), jnp.int32),
        pltpu.VMEM((NH * HR,), jnp.int32),
        pltpu.VMEM((NH * HR,), jnp.int32),
        pltpu.VMEM((NH * HR,), jnp.int32),
        pltpu.VMEM((2, HR, H), jnp.float32),
        pltpu.VMEM((2, HR, H), jnp.float32),
        pltpu.VMEM((HR, H), jnp.float32),
        pltpu.VMEM((HR, H), jnp.float32),
        pltpu.VMEM((2, S, H), jnp.float32),
        pltpu.VMEM((2 * H,), jnp.float32),
        pltpu.SemaphoreType.DMA,
        pltpu.SemaphoreType.DMA,
        pltpu.SemaphoreType.DMA,
        pltpu.SemaphoreType.DMA,
        pltpu.SemaphoreType.DMA,
    ],
)(_sc_body)


def kernel(ans_emb, prev_inds, ocr_embedding, pos_table, type_table,
           ans_ln_w, ans_ln_b, ocr_ln_w, ocr_ln_b, emb_ln_w, emb_ln_b):
    posb, wab = _prep_posb(pos_table, type_table, emb_ln_w, emb_ln_b,
                           ans_ln_b, ocr_ln_b, ans_ln_w, ocr_ln_w)
    ocr2 = jnp.transpose(ocr_embedding, (1, 0, 2)).reshape(OCR * B, H)
    prev_t = jnp.transpose(prev_inds)
    return _sc_kernel(ans_emb, prev_t, ocr2, posb, wab)
